# baseline (speedup 1.0000x reference)
import jax
import jax.numpy as jnp
from jax import lax
from jax.experimental import pallas as pl
from jax.experimental.pallas import tpu as pltpu
from jax.experimental.pallas import tpu_sc as plsc

M = 4096
D = 256
L = 16
NC = 2
NS = 16
NW = NC * NS
R = 512
RPW = R // NW
B = 256
NB = (M - R) // B


def _sc_update(x_hbm, emb_hbm, adj_hbm, pi_hbm, pf_hbm, out_hbm,
               x_v, pi_v, pf_v, wrow_v, buf, sem_g, sem_i, sem_s):
    wid = lax.axis_index("s") * NC + lax.axis_index("c")
    base = wid * RPW
    pltpu.sync_copy(pi_hbm, pi_v)
    gat = pltpu.async_copy(adj_hbm.at[pi_v.at[pl.ds(0, 1)]], wrow_v, sem_g)
    emb_in = pltpu.async_copy(emb_hbm.at[pl.ds(base, RPW)], buf, sem_i)
    pf_in = pltpu.async_copy(pf_hbm, pf_v, sem_s)
    x_in = pltpu.async_copy(x_hbm, x_v, sem_s)
    idxv = pi_v[pl.ds(0, L)]
    pf_in.wait()
    x_in.wait()
    lrv = pf_v[pl.ds(0, L)]
    gat.wait()
    w16 = wrow_v[0, pl.ds(base, L)]
    rows = base + lax.broadcasted_iota(jnp.int32, (L,), 0)
    c16 = lrv * jnp.where(rows == idxv, jnp.float32(1.0),
                          jnp.where(w16 > jnp.float32(0.0), w16,
                                    jnp.float32(0.0)))
    xs = [x_v[pl.ds(k * L, L)] for k in range(D // L)]
    emb_in.wait()
    for t in range(RPW):
        cb = jnp.full((L,), c16[t], jnp.float32)
        for k in range(D // L):
            e = buf[t, pl.ds(k * L, L)]
            buf[t, pl.ds(k * L, L)] = e + cb * (xs[k] - e)
    pltpu.sync_copy(buf, out_hbm.at[pl.ds(base, RPW)])


def _tc_update(emb_ref, wrow_ref, x_ref, lr_ref, idx_ref, o_ref):
    i = pl.program_id(0)
    e = emb_ref[...]
    w = jnp.reshape(wrow_ref[...], (B, 1))
    xv = x_ref[...]
    lr = lr_ref[0, 0]
    idxi = idx_ref[0, 0]
    rows = R + i * B + lax.broadcasted_iota(jnp.int32, (B, 1), 0)
    c = lr * jnp.where(rows == idxi, jnp.float32(1.0),
                       jnp.where(w > jnp.float32(0.0), w, jnp.float32(0.0)))
    o_ref[...] = e + c * (xv - e)


def kernel(x, embedding_to_map, embedding_to_map_adj, iter, idx, max_iter):
    lr = jnp.float32(0.1) * (jnp.float32(1.0)
                             - jnp.float32(iter) / jnp.float32(max_iter))
    idx32 = jnp.asarray(idx, jnp.int32)
    p_idx = jnp.full((L,), idx32, jnp.int32)
    p_lr = jnp.full((L,), lr, jnp.float32)

    mesh = plsc.VectorSubcoreMesh(core_axis_name="c", subcore_axis_name="s")
    sc_som = pl.kernel(
        _sc_update,
        out_type=jax.ShapeDtypeStruct((R, D), jnp.float32),
        mesh=mesh,
        scratch_types=[
            pltpu.VMEM((D,), jnp.float32),
            pltpu.VMEM((L,), jnp.int32),
            pltpu.VMEM((L,), jnp.float32),
            pltpu.VMEM((1, M), jnp.float32),
            pltpu.VMEM((RPW, D), jnp.float32),
            pltpu.SemaphoreType.DMA,
            pltpu.SemaphoreType.DMA,
            pltpu.SemaphoreType.DMA,
        ],
    )
    sc_out = sc_som(x, embedding_to_map, embedding_to_map_adj, p_idx, p_lr)

    w_row = lax.dynamic_slice(embedding_to_map_adj, (idx32, jnp.int32(0)),
                              (1, M))
    x2 = jnp.reshape(x, (1, D))
    lr_arr = jnp.full((1, 1), lr, jnp.float32)
    idx_arr = jnp.full((1, 1), idx32, jnp.int32)
    tc_out = pl.pallas_call(
        _tc_update,
        grid=(NB,),
        in_specs=[
            pl.BlockSpec((B, D), lambda i: (R // B + i, 0)),
            pl.BlockSpec((1, B), lambda i: (0, R // B + i)),
            pl.BlockSpec((1, D), lambda i: (0, 0)),
            pl.BlockSpec((1, 1), lambda i: (0, 0)),
            pl.BlockSpec((1, 1), lambda i: (0, 0)),
        ],
        out_specs=pl.BlockSpec((B, D), lambda i: (R // B + i, 0)),
        out_shape=jax.ShapeDtypeStruct((M, D), jnp.float32),
    )(embedding_to_map, w_row, x2, lr_arr, idx_arr)

    return lax.dynamic_update_slice(tc_out, sc_out, (0, 0))


# speedup vs baseline: 1.0961x; 1.0961x over previous
import jax
import jax.numpy as jnp
from jax import lax
from jax.experimental import pallas as pl
from jax.experimental.pallas import tpu as pltpu
from jax.experimental.pallas import tpu_sc as plsc

M = 4096
D = 256
L = 16
NC = 2
NS = 16
NW = NC * NS
RPW = M // NW
CH = 16
NCH = RPW // CH
PAIRS = NCH // 2


def _som_update(x_hbm, emb_hbm, adj_hbm, pi_hbm, pf_hbm, out_hbm,
                x_v, pi_v, pf_v, wrow_v, buf_a, buf_b,
                sem_g, sem_ai, sem_bi, sem_ao, sem_bo, sem_s):
    wid = lax.axis_index("s") * NC + lax.axis_index("c")
    base = wid * RPW
    pltpu.sync_copy(pi_hbm, pi_v)
    gat = pltpu.async_copy(adj_hbm.at[pi_v.at[pl.ds(0, 1)]], wrow_v, sem_g)
    pltpu.async_copy(emb_hbm.at[pl.ds(base, CH)], buf_a, sem_ai)
    pltpu.async_copy(emb_hbm.at[pl.ds(base + CH, CH)], buf_b, sem_bi)
    pf_in = pltpu.async_copy(pf_hbm, pf_v, sem_s)
    x_in = pltpu.async_copy(x_hbm, x_v, sem_s)
    idxv = pi_v[pl.ds(0, L)]
    pf_in.wait()
    x_in.wait()
    lrv = pf_v[pl.ds(0, L)]
    gat.wait()
    xs = [x_v[pl.ds(k * L, L)] for k in range(D // L)]

    def process(buf, r0):
        w16 = wrow_v[0, pl.ds(r0, L)]
        rows = r0 + lax.broadcasted_iota(jnp.int32, (L,), 0)
        c16 = lrv * jnp.where(rows == idxv, jnp.float32(1.0),
                              jnp.where(w16 > jnp.float32(0.0), w16,
                                        jnp.float32(0.0)))
        for t in range(L):
            cb = jnp.full((L,), c16[t], jnp.float32)
            for k in range(D // L):
                e = buf[t, pl.ds(k * L, L)]
                buf[t, pl.ds(k * L, L)] = e + cb * (xs[k] - e)

    def pair(it, carry):
        a0 = base + (2 * it) * CH
        b0 = a0 + CH
        pltpu.make_async_copy(emb_hbm.at[pl.ds(a0, CH)], buf_a, sem_ai).wait()
        process(buf_a, a0)
        pltpu.async_copy(buf_a, out_hbm.at[pl.ds(a0, CH)], sem_ao)
        pltpu.make_async_copy(emb_hbm.at[pl.ds(b0, CH)], buf_b, sem_bi).wait()
        process(buf_b, b0)
        pltpu.async_copy(buf_b, out_hbm.at[pl.ds(b0, CH)], sem_bo)

        @pl.when(it < PAIRS - 1)
        def _refill():
            pltpu.make_async_copy(buf_a, out_hbm.at[pl.ds(a0, CH)],
                                  sem_ao).wait()
            pltpu.async_copy(emb_hbm.at[pl.ds(a0 + 2 * CH, CH)], buf_a, sem_ai)
            pltpu.make_async_copy(buf_b, out_hbm.at[pl.ds(b0, CH)],
                                  sem_bo).wait()
            pltpu.async_copy(emb_hbm.at[pl.ds(b0 + 2 * CH, CH)], buf_b, sem_bi)

        return carry

    lax.fori_loop(0, PAIRS, pair, 0)
    last_a = base + (NCH - 2) * CH
    last_b = base + (NCH - 1) * CH
    pltpu.make_async_copy(buf_a, out_hbm.at[pl.ds(last_a, CH)], sem_ao).wait()
    pltpu.make_async_copy(buf_b, out_hbm.at[pl.ds(last_b, CH)], sem_bo).wait()


def kernel(x, embedding_to_map, embedding_to_map_adj, iter, idx, max_iter):
    lr = jnp.float32(0.1) * (jnp.float32(1.0)
                             - jnp.float32(iter) / jnp.float32(max_iter))
    idx32 = jnp.asarray(idx, jnp.int32)
    p_idx = jnp.full((L,), idx32, jnp.int32)
    p_lr = jnp.full((L,), lr, jnp.float32)
    mesh = plsc.VectorSubcoreMesh(core_axis_name="c", subcore_axis_name="s")
    som = pl.kernel(
        _som_update,
        out_type=jax.ShapeDtypeStruct((M, D), jnp.float32),
        mesh=mesh,
        scratch_types=[
            pltpu.VMEM((D,), jnp.float32),
            pltpu.VMEM((L,), jnp.int32),
            pltpu.VMEM((L,), jnp.float32),
            pltpu.VMEM((1, M), jnp.float32),
            pltpu.VMEM((CH, D), jnp.float32),
            pltpu.VMEM((CH, D), jnp.float32),
            pltpu.SemaphoreType.DMA,
            pltpu.SemaphoreType.DMA,
            pltpu.SemaphoreType.DMA,
            pltpu.SemaphoreType.DMA,
            pltpu.SemaphoreType.DMA,
            pltpu.SemaphoreType.DMA,
        ],
    )
    return som(x, embedding_to_map, embedding_to_map_adj, p_idx, p_lr)


# speedup vs baseline: 1.1313x; 1.0321x over previous
import jax
import jax.numpy as jnp
from jax import lax
from jax.experimental import pallas as pl
from jax.experimental.pallas import tpu as pltpu
from jax.experimental.pallas import tpu_sc as plsc

M = 4096
D = 256
L = 16
NC = 2
NS = 16
NW = NC * NS
RPW = M // NW
CH = 16
NCH = RPW // CH
PAIRS = NCH // 2


def _som_update(x_hbm, emb_hbm, adj_hbm, pi_hbm, pf_hbm, out_hbm,
                x_v, pi_v, pf_v, wrow_v, buf_a, buf_b,
                sem_g, sem_ai, sem_bi, sem_ao, sem_bo):
    wid = lax.axis_index("s") * NC + lax.axis_index("c")
    base = wid * RPW
    pltpu.sync_copy(pi_hbm, pi_v)
    gat = pltpu.async_copy(adj_hbm.at[pi_v.at[pl.ds(0, 1)]], wrow_v, sem_g)
    pltpu.sync_copy(pf_hbm, pf_v)
    pltpu.sync_copy(x_hbm, x_v)
    pltpu.async_copy(emb_hbm.at[pl.ds(base, CH)], buf_a, sem_ai)
    pltpu.async_copy(emb_hbm.at[pl.ds(base + CH, CH)], buf_b, sem_bi)
    idxv = pi_v[pl.ds(0, L)]
    lrv = pf_v[pl.ds(0, L)]
    gat.wait()
    xs = [x_v[pl.ds(k * L, L)] for k in range(D // L)]

    def process(buf, r0):
        w16 = wrow_v[0, pl.ds(r0, L)]
        rows = r0 + lax.broadcasted_iota(jnp.int32, (L,), 0)
        c16 = lrv * jnp.where(rows == idxv, jnp.float32(1.0),
                              jnp.where(w16 > jnp.float32(0.0), w16,
                                        jnp.float32(0.0)))
        for t in range(L):
            cb = jnp.full((L,), c16[t], jnp.float32)
            for k in range(D // L):
                e = buf[t, pl.ds(k * L, L)]
                buf[t, pl.ds(k * L, L)] = e + cb * (xs[k] - e)

    def pair(it, carry):
        a0 = base + (2 * it) * CH
        b0 = a0 + CH
        pltpu.make_async_copy(emb_hbm.at[pl.ds(a0, CH)], buf_a, sem_ai).wait()
        process(buf_a, a0)
        pltpu.async_copy(buf_a, out_hbm.at[pl.ds(a0, CH)], sem_ao)
        pltpu.make_async_copy(emb_hbm.at[pl.ds(b0, CH)], buf_b, sem_bi).wait()
        process(buf_b, b0)
        pltpu.async_copy(buf_b, out_hbm.at[pl.ds(b0, CH)], sem_bo)

        @pl.when(it < PAIRS - 1)
        def _refill():
            pltpu.make_async_copy(buf_a, out_hbm.at[pl.ds(a0, CH)],
                                  sem_ao).wait()
            pltpu.async_copy(emb_hbm.at[pl.ds(a0 + 2 * CH, CH)], buf_a, sem_ai)
            pltpu.make_async_copy(buf_b, out_hbm.at[pl.ds(b0, CH)],
                                  sem_bo).wait()
            pltpu.async_copy(emb_hbm.at[pl.ds(b0 + 2 * CH, CH)], buf_b, sem_bi)

        return carry

    lax.fori_loop(0, PAIRS, pair, 0)
    last_a = base + (NCH - 2) * CH
    last_b = base + (NCH - 1) * CH
    pltpu.make_async_copy(buf_a, out_hbm.at[pl.ds(last_a, CH)], sem_ao).wait()
    pltpu.make_async_copy(buf_b, out_hbm.at[pl.ds(last_b, CH)], sem_bo).wait()


def kernel(x, embedding_to_map, embedding_to_map_adj, iter, idx, max_iter):
    lr = jnp.float32(0.1) * (jnp.float32(1.0)
                             - jnp.float32(iter) / jnp.float32(max_iter))
    idx32 = jnp.asarray(idx, jnp.int32)
    p_idx = jnp.full((L,), idx32, jnp.int32)
    p_lr = jnp.full((L,), lr, jnp.float32)
    mesh = plsc.VectorSubcoreMesh(core_axis_name="c", subcore_axis_name="s")
    som = pl.kernel(
        _som_update,
        out_type=jax.ShapeDtypeStruct((M, D), jnp.float32),
        mesh=mesh,
        scratch_types=[
            pltpu.VMEM((D,), jnp.float32),
            pltpu.VMEM((L,), jnp.int32),
            pltpu.VMEM((L,), jnp.float32),
            pltpu.VMEM((1, M), jnp.float32),
            pltpu.VMEM((CH, D), jnp.float32),
            pltpu.VMEM((CH, D), jnp.float32),
            pltpu.SemaphoreType.DMA,
            pltpu.SemaphoreType.DMA,
            pltpu.SemaphoreType.DMA,
            pltpu.SemaphoreType.DMA,
            pltpu.SemaphoreType.DMA,
        ],
    )
    return som(x, embedding_to_map, embedding_to_map_adj, p_idx, p_lr)


# speedup vs baseline: 1.1593x; 1.0248x over previous
import jax
import jax.numpy as jnp
from jax import lax
from jax.experimental import pallas as pl
from jax.experimental.pallas import tpu as pltpu
from jax.experimental.pallas import tpu_sc as plsc

M = 4096
D = 256
L = 16
NC = 2
NS = 16
NW = NC * NS
RPW = M // NW
CH = 16
NCH = RPW // CH
PAIRS = NCH // 2


def _som_update(x_hbm, emb_hbm, adj_hbm, pi_hbm, pf_hbm, out_hbm,
                x_v, pi_v, pf_v, wrow_v, buf_a, buf_b,
                sem_g, sem_ai, sem_bi, sem_ao, sem_bo):
    wid = lax.axis_index("s") * NC + lax.axis_index("c")
    base = wid * RPW
    pltpu.sync_copy(pi_hbm, pi_v)
    gat = pltpu.async_copy(adj_hbm.at[pi_v.at[pl.ds(0, 1)]], wrow_v, sem_g)
    pltpu.sync_copy(pf_hbm, pf_v)
    pltpu.sync_copy(x_hbm, x_v)
    pltpu.async_copy(emb_hbm.at[pl.ds(base, CH)], buf_a, sem_ai)
    pltpu.async_copy(emb_hbm.at[pl.ds(base + CH, CH)], buf_b, sem_bi)
    idxv = pi_v[pl.ds(0, L)]
    lrv = pf_v[pl.ds(0, L)]
    gat.wait()
    xs = [x_v[pl.ds(k * L, L)] for k in range(D // L)]

    def process(buf, r0):
        w16 = wrow_v[0, pl.ds(r0, L)]
        rows = r0 + lax.broadcasted_iota(jnp.int32, (L,), 0)
        c16 = lrv * jnp.where(rows == idxv, jnp.float32(1.0),
                              jnp.where(w16 > jnp.float32(0.0), w16,
                                        jnp.float32(0.0)))
        def row_body(t, carry):
            cb = lax.gather(
                c16, jnp.full((L, 1), t, jnp.int32),
                lax.GatherDimensionNumbers(offset_dims=(),
                                           collapsed_slice_dims=(0,),
                                           start_index_map=(0,)),
                (1,), mode=lax.GatherScatterMode.PROMISE_IN_BOUNDS)
            for k in range(D // L):
                e = buf[t, pl.ds(k * L, L)]
                buf[t, pl.ds(k * L, L)] = e + cb * (xs[k] - e)
            return carry

        lax.fori_loop(0, L, row_body, 0)

    def pair(it, carry):
        a0 = base + (2 * it) * CH
        b0 = a0 + CH
        pltpu.make_async_copy(emb_hbm.at[pl.ds(a0, CH)], buf_a, sem_ai).wait()
        process(buf_a, a0)
        pltpu.async_copy(buf_a, out_hbm.at[pl.ds(a0, CH)], sem_ao)
        pltpu.make_async_copy(emb_hbm.at[pl.ds(b0, CH)], buf_b, sem_bi).wait()
        process(buf_b, b0)
        pltpu.async_copy(buf_b, out_hbm.at[pl.ds(b0, CH)], sem_bo)

        @pl.when(it < PAIRS - 1)
        def _refill():
            pltpu.make_async_copy(buf_a, out_hbm.at[pl.ds(a0, CH)],
                                  sem_ao).wait()
            pltpu.async_copy(emb_hbm.at[pl.ds(a0 + 2 * CH, CH)], buf_a, sem_ai)
            pltpu.make_async_copy(buf_b, out_hbm.at[pl.ds(b0, CH)],
                                  sem_bo).wait()
            pltpu.async_copy(emb_hbm.at[pl.ds(b0 + 2 * CH, CH)], buf_b, sem_bi)

        return carry

    lax.fori_loop(0, PAIRS, pair, 0)
    last_a = base + (NCH - 2) * CH
    last_b = base + (NCH - 1) * CH
    pltpu.make_async_copy(buf_a, out_hbm.at[pl.ds(last_a, CH)], sem_ao).wait()
    pltpu.make_async_copy(buf_b, out_hbm.at[pl.ds(last_b, CH)], sem_bo).wait()


def kernel(x, embedding_to_map, embedding_to_map_adj, iter, idx, max_iter):
    lr = jnp.float32(0.1) * (jnp.float32(1.0)
                             - jnp.float32(iter) / jnp.float32(max_iter))
    idx32 = jnp.asarray(idx, jnp.int32)
    p_idx = jnp.full((L,), idx32, jnp.int32)
    p_lr = jnp.full((L,), lr, jnp.float32)
    mesh = plsc.VectorSubcoreMesh(core_axis_name="c", subcore_axis_name="s")
    som = pl.kernel(
        _som_update,
        out_type=jax.ShapeDtypeStruct((M, D), jnp.float32),
        mesh=mesh,
        scratch_types=[
            pltpu.VMEM((D,), jnp.float32),
            pltpu.VMEM((L,), jnp.int32),
            pltpu.VMEM((L,), jnp.float32),
            pltpu.VMEM((1, M), jnp.float32),
            pltpu.VMEM((CH, D), jnp.float32),
            pltpu.VMEM((CH, D), jnp.float32),
            pltpu.SemaphoreType.DMA,
            pltpu.SemaphoreType.DMA,
            pltpu.SemaphoreType.DMA,
            pltpu.SemaphoreType.DMA,
            pltpu.SemaphoreType.DMA,
        ],
    )
    return som(x, embedding_to_map, embedding_to_map_adj, p_idx, p_lr)
